# baseline (speedup 1.0000x reference)
import functools

import jax
import jax.numpy as jnp
from jax import lax
from jax.experimental import pallas as pl
from jax.experimental.pallas import tpu as pltpu
from jax.experimental.pallas import tpu_sc as plsc

_REG = 0.1
_NUM_ITER = 8
_STOP_THR = 1e-6

_NC = 2
_NSUB = 16
_NW = _NC * _NSUB
_CHUNK = 128
_NSUBCH = 8

_HI = jax.lax.Precision.HIGHEST


def _sc_segsum2(xa, srca, dsta, xb, srcb, dstb):
    na, d = xa.shape
    nb = xb.shape[0]
    nmax = max(na, nb)
    nbufs = 2
    n_sub = _NSUBCH

    mesh = plsc.VectorSubcoreMesh(core_axis_name="c", subcore_axis_name="s",
                                  num_cores=_NC, num_subcores=_NSUB)

    def body(xa_hbm, srca_hbm, dsta_hbm, xb_hbm, srcb_hbm, dstb_hbm,
             zeros_a_hbm, zeros_b_hbm, outa_hbm, outb_hbm,
             idx_s_v, idx_d_v, rows0_v, rows1_v, rows2_v,
             acc_sh, gsem, ssem):
        rows_bufs = (rows0_v, rows1_v, rows2_v)
        c = lax.axis_index("c")
        s = lax.axis_index("s")

        def phase(x_hbm, src_hbm, dst_hbm, zeros_hbm, out_hbm, n, e):
            rpt = n // _NSUB
            epw = e // _NW
            r0 = s * rpt
            pltpu.sync_copy(zeros_hbm, acc_sh.at[pl.ds(r0, rpt)])
            plsc.subcore_barrier()

            w = c * _NSUB + s
            n_outer = epw // (n_sub * _CHUNK)
            blk_base = w * n_outer

            def outer(j, carry):
                pltpu.sync_copy(src_hbm.at[blk_base + j], idx_s_v)
                pltpu.sync_copy(dst_hbm.at[blk_base + j], idx_d_v)
                gd = [None] * n_sub
                sd = [None] * n_sub
                for k in range(min(nbufs - 1, n_sub)):
                    gd[k] = pltpu.async_copy(x_hbm.at[idx_s_v.at[k]],
                                             rows_bufs[k % nbufs], gsem)
                for k in range(n_sub):
                    buf = rows_bufs[k % nbufs]
                    gd[k].wait()
                    sd[k] = pltpu.async_copy(buf, acc_sh.at[idx_d_v.at[k]],
                                             ssem, add=True)
                    kn = k + nbufs - 1
                    if kn < n_sub:
                        if kn - nbufs >= 0:
                            sd[kn - nbufs].wait()
                        gd[kn] = pltpu.async_copy(
                            x_hbm.at[idx_s_v.at[kn]],
                            rows_bufs[kn % nbufs], gsem)
                for k in range(max(0, n_sub - nbufs), n_sub):
                    if sd[k] is not None:
                        sd[k].wait()
                return carry

            lax.fori_loop(0, n_outer, outer, 0)
            plsc.subcore_barrier()
            pltpu.sync_copy(acc_sh.at[pl.ds(r0, rpt)],
                            out_hbm.at[pl.ds(c * n + r0, rpt)])
            plsc.subcore_barrier()

        ea = srca_hbm.shape[0] * _NSUBCH * _CHUNK
        eb = srcb_hbm.shape[0] * _NSUBCH * _CHUNK
        phase(xa_hbm, srca_hbm, dsta_hbm, zeros_a_hbm, outa_hbm, na, ea)
        phase(xb_hbm, srcb_hbm, dstb_hbm, zeros_b_hbm, outb_hbm, nb, eb)

    kfn = pl.kernel(
        body,
        out_type=(jax.ShapeDtypeStruct((2 * na, d), jnp.float32),
                  jax.ShapeDtypeStruct((2 * nb, d), jnp.float32)),
        mesh=mesh,
        scratch_types=[
            pltpu.VMEM((_NSUBCH, _CHUNK), jnp.int32),
            pltpu.VMEM((_NSUBCH, _CHUNK), jnp.int32),
            pltpu.VMEM((_CHUNK, d), jnp.float32),
            pltpu.VMEM((_CHUNK, d), jnp.float32),
            pltpu.VMEM((_CHUNK, d), jnp.float32),
            pltpu.VMEM_SHARED((nmax, d), jnp.float32),
            pltpu.SemaphoreType.DMA,
            pltpu.SemaphoreType.DMA,
        ],
    )
    zeros_a = jnp.zeros((na // _NSUB, d), jnp.float32)
    zeros_b = jnp.zeros((nb // _NSUB, d), jnp.float32)
    outa, outb = kfn(xa, srca, dsta, xb, srcb, dstb, zeros_a, zeros_b)
    return outa.reshape(2, na, d), outb.reshape(2, nb, d)


def _tc_h1(xg, parts, w_rel, b_rel, w_root, blk):
    N, din = xg.shape
    da = parts.shape[2]
    h = w_rel.shape[1]
    grid = N // blk

    def body(x_ref, p_ref, wrel_ref, brel_ref, wroot_ref, out_ref):
        agg = p_ref[0] + p_ref[1]
        acc = lax.dot_general(agg, wrel_ref[...], (((1,), (0,)), ((), ())),
                              precision=_HI, preferred_element_type=jnp.float32)
        acc = acc + lax.dot_general(x_ref[...], wroot_ref[...],
                                    (((1,), (0,)), ((), ())),
                                    precision=_HI,
                                    preferred_element_type=jnp.float32)
        out_ref[...] = jnp.maximum(acc + brel_ref[...], 0.0)

    return pl.pallas_call(
        body,
        grid=(grid,),
        in_specs=[
            pl.BlockSpec((blk, din), lambda p: (p, 0)),
            pl.BlockSpec((2, blk, da), lambda p: (0, p, 0)),
            pl.BlockSpec((da, h), lambda p: (0, 0)),
            pl.BlockSpec((1, h), lambda p: (0, 0)),
            pl.BlockSpec((din, h), lambda p: (0, 0)),
        ],
        out_specs=pl.BlockSpec((blk, h), lambda p: (p, 0)),
        out_shape=jax.ShapeDtypeStruct((N, h), jnp.float32),
    )(xg, parts, w_rel, b_rel.reshape(1, h), w_root)


def _mm(a, b):
    return lax.dot_general(a, b, (((1,), (0,)), ((), ())),
                           precision=_HI, preferred_element_type=jnp.float32)


def _pair_kernel(ns, nt, din, h, de, npairs):
    def body(xs_ref, h1s_ref, a2s_ref, xt_ref, h1t_ref, a2t_ref,
             wrel2_ref, brel2_ref, wroot2_ref,
             wex_ref, weh1_ref, weh2_ref, bemb_ref, virt_ref,
             m_ref, err_ref, g1_ref, g8_ref):
        brel2 = brel2_ref[...]
        bemb = bemb_ref[...]

        def embed(x, h1, aggp):
            agg = aggp[0] + aggp[1]
            h2 = jnp.maximum(
                _mm(agg, wrel2_ref[...]) + brel2 + _mm(h1, wroot2_ref[...]),
                0.0)
            pre = (_mm(x, wex_ref[...]) + _mm(h1, weh1_ref[...])
                   + _mm(h2, weh2_ref[...]) + bemb)
            nrm = jnp.sqrt(jnp.sum(pre * pre, axis=1, keepdims=True))
            return pre / jnp.maximum(nrm, 1e-12)

        es = embed(xs_ref[...], h1s_ref[...], a2s_ref[...])
        et = embed(xt_ref[...], h1t_ref[...], a2t_ref[...])
        virt = virt_ref[...]
        vn = virt / jnp.maximum(
            jnp.sqrt(jnp.sum(virt * virt, axis=1, keepdims=True)), 1e-12)

        a2 = jnp.sum(es * es, axis=1, keepdims=True)
        b2 = jnp.sum(et * et, axis=1)
        g = lax.dot_general(es, et, (((1,), (1,)), ((), ())),
                            precision=_HI, preferred_element_type=jnp.float32)
        d2 = a2 + b2[None, :] - 2.0 * g
        c_real = jnp.sqrt(jnp.clip(d2, 1e-12, None))
        gv = lax.dot_general(es, vn, (((1,), (1,)), ((), ())),
                             precision=_HI, preferred_element_type=jnp.float32)
        d2v = a2 + jnp.sum(vn * vn) - 2.0 * gv
        cv = jnp.sqrt(jnp.clip(d2v, 1e-12, None))
        m = jnp.concatenate([c_real, jnp.broadcast_to(cv, (ns, ns - nt))],
                            axis=1)
        m_ref[...] = m

        k = jnp.exp(m * (-1.0 / _REG))

        def upd(u):
            ktu = lax.dot_general(u, k, (((1,), (0,)), ((), ())),
                                  precision=_HI,
                                  preferred_element_type=jnp.float32)
            vv = 1.0 / ktu
            kv = lax.dot_general(vv, k, (((1,), (1,)), ((), ())),
                                 precision=_HI,
                                 preferred_element_type=jnp.float32)
            return 1.0 / kv, vv

        u0 = jnp.full((1, ns), 1.0 / ns, jnp.float32)
        u1, v1 = upd(u0)
        ktu1 = lax.dot_general(u1, k, (((1,), (0,)), ((), ())),
                               precision=_HI, preferred_element_type=jnp.float32)
        tmp2 = v1 * ktu1
        err = jnp.sqrt(jnp.sum((tmp2 - 1.0) ** 2))

        km = k * m

        def geds(u, vv):
            kmv = lax.dot_general(vv, km, (((1,), (1,)), ((), ())),
                                  precision=_HI,
                                  preferred_element_type=jnp.float32)
            return jnp.sum(u * kmv)

        g1 = geds(u1, v1)
        u, vv = u1, v1
        for _ in range(1, _NUM_ITER):
            u, vv = upd(u)
        g8 = geds(u, vv)

        err_ref[...] = jnp.full((1, 1, 128), err, jnp.float32)
        g1_ref[...] = jnp.full((1, 1, 128), g1, jnp.float32)
        g8_ref[...] = jnp.full((1, 1, 128), g8, jnp.float32)

    return body


def _tc_pairs(xs, h1s, a2s, xt, h1t, a2t,
              w_rel2, b_rel2, w_root2, we_x, we_h1, we_h2, b_emb, virt,
              npairs, ns, nt):
    din = xs.shape[1]
    h = h1s.shape[1]
    de = we_x.shape[1]
    body = _pair_kernel(ns, nt, din, h, de, npairs)
    m, err, g1, g8 = pl.pallas_call(
        body,
        grid=(npairs,),
        in_specs=[
            pl.BlockSpec((ns, din), lambda p: (p, 0)),
            pl.BlockSpec((ns, h), lambda p: (p, 0)),
            pl.BlockSpec((2, ns, h), lambda p: (0, p, 0)),
            pl.BlockSpec((nt, din), lambda p: (p, 0)),
            pl.BlockSpec((nt, h), lambda p: (p, 0)),
            pl.BlockSpec((2, nt, h), lambda p: (0, p, 0)),
            pl.BlockSpec((h, h), lambda p: (0, 0)),
            pl.BlockSpec((1, h), lambda p: (0, 0)),
            pl.BlockSpec((h, h), lambda p: (0, 0)),
            pl.BlockSpec((din, de), lambda p: (0, 0)),
            pl.BlockSpec((h, de), lambda p: (0, 0)),
            pl.BlockSpec((h, de), lambda p: (0, 0)),
            pl.BlockSpec((1, de), lambda p: (0, 0)),
            pl.BlockSpec((1, de), lambda p: (0, 0)),
        ],
        out_specs=[
            pl.BlockSpec((ns, ns), lambda p: (p, 0)),
            pl.BlockSpec((1, 1, 128), lambda p: (p, 0, 0)),
            pl.BlockSpec((1, 1, 128), lambda p: (p, 0, 0)),
            pl.BlockSpec((1, 1, 128), lambda p: (p, 0, 0)),
        ],
        out_shape=[
            jax.ShapeDtypeStruct((npairs * ns, ns), jnp.float32),
            jax.ShapeDtypeStruct((npairs, 1, 128), jnp.float32),
            jax.ShapeDtypeStruct((npairs, 1, 128), jnp.float32),
            jax.ShapeDtypeStruct((npairs, 1, 128), jnp.float32),
        ],
    )(xs, h1s, a2s, xt, h1t, a2t,
      w_rel2, b_rel2.reshape(1, h), w_root2,
      we_x, we_h1, we_h2, b_emb.reshape(1, de), virt.reshape(1, de))
    return m, err[:, 0, 0], g1[:, 0, 0], g8[:, 0, 0]


def kernel(x_s, edge_index_s, edge_attr_s, x_t, edge_index_t, edge_attr_t,
           x_s_batch, len_s, len_t, W_rel1, b_rel1, W_root1,
           W_rel2, b_rel2, W_root2, W_emb, b_emb, virtual_embedding):
    npairs = len_s.shape[0]
    n_s, din = x_s.shape
    n_t = x_t.shape[0]
    ns = n_s // npairs
    nt = n_t // npairs
    h = W_rel1.shape[1]
    de = W_emb.shape[1]

    blk = _NSUBCH * _CHUNK
    e_s = edge_index_s.shape[1]
    e_t = edge_index_t.shape[1]
    e_t_pad = ((e_t + blk * _NW - 1) // (blk * _NW)) * (blk * _NW)
    pad_t = e_t_pad - e_t
    src_s = edge_index_s[0].reshape(-1, _NSUBCH, _CHUNK)
    dst_s = edge_index_s[1].reshape(-1, _NSUBCH, _CHUNK)
    src_t = jnp.concatenate(
        [edge_index_t[0], jnp.zeros((pad_t,), edge_index_t.dtype)]
    ).reshape(-1, _NSUBCH, _CHUNK)
    dst_t = jnp.concatenate(
        [edge_index_t[1], jnp.full((pad_t,), n_t, edge_index_t.dtype)]
    ).reshape(-1, _NSUBCH, _CHUNK)

    pad = 128 - din
    x_s_pad = jnp.pad(x_s, ((0, 0), (0, pad)))
    x_t_pad = jnp.pad(x_t, ((0, 0), (0, pad)))
    w_rel1_pad = jnp.pad(W_rel1, ((0, pad), (0, 0)))

    agg1_s, agg1_t = _sc_segsum2(x_s_pad, src_s, dst_s, x_t_pad, src_t, dst_t)
    h1s = _tc_h1(x_s, agg1_s, w_rel1_pad, b_rel1, W_root1, ns)
    h1t = _tc_h1(x_t, agg1_t, w_rel1_pad, b_rel1, W_root1, nt)

    agg2_s, agg2_t = _sc_segsum2(h1s, src_s, dst_s, h1t, src_t, dst_t)

    we_x = W_emb[:din]
    we_h1 = W_emb[din:din + h]
    we_h2 = W_emb[din + h:]

    m, err, g1, g8 = _tc_pairs(
        x_s, h1s, agg2_s, x_t, h1t, agg2_t,
        W_rel2, b_rel2, W_root2, we_x, we_h1, we_h2, b_emb,
        virtual_embedding, npairs, ns, nt)

    edit_costs2 = m.reshape(npairs, ns, ns)
    done = jnp.all(err < _STOP_THR)
    geds = jnp.where(done, g1, g8)
    geds2 = geds / (len_s + len_t).astype(jnp.float32)
    return (edit_costs2, geds2)


# speedup vs baseline: 1.0006x; 1.0006x over previous
import functools

import jax
import jax.numpy as jnp
from jax import lax
from jax.experimental import pallas as pl
from jax.experimental.pallas import tpu as pltpu
from jax.experimental.pallas import tpu_sc as plsc

_REG = 0.1
_NUM_ITER = 8
_STOP_THR = 1e-6

_NC = 2
_NSUB = 16
_NW = _NC * _NSUB
_CHUNK = 128
_NSUBCH = 8

_HI = jax.lax.Precision.HIGHEST


def _sc_segsum2(xa, srca, dsta, xb, srcb, dstb):
    na, d = xa.shape
    nb = xb.shape[0]
    nmax = max(na, nb)
    nbufs = 2
    n_sub = _NSUBCH

    mesh = plsc.VectorSubcoreMesh(core_axis_name="c", subcore_axis_name="s",
                                  num_cores=_NC, num_subcores=_NSUB)

    def body(xa_hbm, srca_hbm, dsta_hbm, xb_hbm, srcb_hbm, dstb_hbm,
             zeros_a_hbm, zeros_b_hbm, outa_hbm, outb_hbm,
             idx_s_v, idx_d_v, rows0_v, rows1_v, rows2_v,
             acc_sh, gsem, ssem):
        rows_bufs = (rows0_v, rows1_v, rows2_v)
        c = lax.axis_index("c")
        s = lax.axis_index("s")

        def phase(x_hbm, src_hbm, dst_hbm, zeros_hbm, out_hbm, n, e):
            rpt = n // _NSUB
            epw = e // _NW
            r0 = s * rpt
            pltpu.sync_copy(zeros_hbm, acc_sh.at[pl.ds(r0, rpt)])
            plsc.subcore_barrier()

            w = c * _NSUB + s
            n_outer = epw // (n_sub * _CHUNK)
            blk_base = w * n_outer

            def outer(j, carry):
                pltpu.sync_copy(src_hbm.at[blk_base + j], idx_s_v)
                pltpu.sync_copy(dst_hbm.at[blk_base + j], idx_d_v)
                gd = [None] * n_sub
                sd = [None] * n_sub
                for k in range(min(nbufs - 1, n_sub)):
                    gd[k] = pltpu.async_copy(x_hbm.at[idx_s_v.at[k]],
                                             rows_bufs[k % nbufs], gsem)
                for k in range(n_sub):
                    buf = rows_bufs[k % nbufs]
                    gd[k].wait()
                    sd[k] = pltpu.async_copy(buf, acc_sh.at[idx_d_v.at[k]],
                                             ssem, add=True)
                    kn = k + nbufs - 1
                    if kn < n_sub:
                        if kn - nbufs >= 0:
                            sd[kn - nbufs].wait()
                        gd[kn] = pltpu.async_copy(
                            x_hbm.at[idx_s_v.at[kn]],
                            rows_bufs[kn % nbufs], gsem)
                for k in range(max(0, n_sub - nbufs), n_sub):
                    if sd[k] is not None:
                        sd[k].wait()
                return carry

            lax.fori_loop(0, n_outer, outer, 0)
            plsc.subcore_barrier()
            pltpu.sync_copy(acc_sh.at[pl.ds(r0, rpt)],
                            out_hbm.at[pl.ds(c * n + r0, rpt)])
            plsc.subcore_barrier()

        ea = srca_hbm.shape[0] * _NSUBCH * _CHUNK
        eb = srcb_hbm.shape[0] * _NSUBCH * _CHUNK
        phase(xa_hbm, srca_hbm, dsta_hbm, zeros_a_hbm, outa_hbm, na, ea)
        phase(xb_hbm, srcb_hbm, dstb_hbm, zeros_b_hbm, outb_hbm, nb, eb)

    kfn = pl.kernel(
        body,
        out_type=(jax.ShapeDtypeStruct((2 * na, d), jnp.float32),
                  jax.ShapeDtypeStruct((2 * nb, d), jnp.float32)),
        mesh=mesh,
        scratch_types=[
            pltpu.VMEM((_NSUBCH, _CHUNK), jnp.int32),
            pltpu.VMEM((_NSUBCH, _CHUNK), jnp.int32),
            pltpu.VMEM((_CHUNK, d), jnp.float32),
            pltpu.VMEM((_CHUNK, d), jnp.float32),
            pltpu.VMEM((_CHUNK, d), jnp.float32),
            pltpu.VMEM_SHARED((nmax, d), jnp.float32),
            pltpu.SemaphoreType.DMA,
            pltpu.SemaphoreType.DMA,
        ],
    )
    zeros_a = jnp.zeros((na // _NSUB, d), jnp.float32)
    zeros_b = jnp.zeros((nb // _NSUB, d), jnp.float32)
    outa, outb = kfn(xa, srca, dsta, xb, srcb, dstb, zeros_a, zeros_b)
    return outa.reshape(2, na, d), outb.reshape(2, nb, d)


def _tc_h1(xg, parts, w_rel, b_rel, w_root, blk):
    N, din = xg.shape
    da = parts.shape[2]
    h = w_rel.shape[1]
    grid = N // blk

    def body(x_ref, p_ref, wrel_ref, brel_ref, wroot_ref, out_ref):
        agg = p_ref[0] + p_ref[1]
        acc = lax.dot_general(agg, wrel_ref[...], (((1,), (0,)), ((), ())),
                              precision=_HI, preferred_element_type=jnp.float32)
        acc = acc + lax.dot_general(x_ref[...], wroot_ref[...],
                                    (((1,), (0,)), ((), ())),
                                    precision=_HI,
                                    preferred_element_type=jnp.float32)
        out_ref[...] = jnp.maximum(acc + brel_ref[...], 0.0)

    return pl.pallas_call(
        body,
        grid=(grid,),
        in_specs=[
            pl.BlockSpec((blk, din), lambda p: (p, 0)),
            pl.BlockSpec((2, blk, da), lambda p: (0, p, 0)),
            pl.BlockSpec((da, h), lambda p: (0, 0)),
            pl.BlockSpec((1, h), lambda p: (0, 0)),
            pl.BlockSpec((din, h), lambda p: (0, 0)),
        ],
        out_specs=pl.BlockSpec((blk, h), lambda p: (p, 0)),
        out_shape=jax.ShapeDtypeStruct((N, h), jnp.float32),
    )(xg, parts, w_rel, b_rel.reshape(1, h), w_root)


def _mm(a, b):
    return lax.dot_general(a, b, (((1,), (0,)), ((), ())),
                           precision=_HI, preferred_element_type=jnp.float32)


def _pair_kernel(ns, nt, din, h, de, npairs):
    def body(xs_ref, h1s_ref, a2s_ref, xt_ref, h1t_ref, a2t_ref,
             wrel2_ref, brel2_ref, wroot2_ref,
             wex_ref, weh1_ref, weh2_ref, bemb_ref, virt_ref,
             m_ref, err_ref, g1_ref, g8_ref):
        brel2 = brel2_ref[...]
        bemb = bemb_ref[...]

        def embed(x, h1, aggp):
            agg = aggp[0] + aggp[1]
            h2 = jnp.maximum(
                _mm(agg, wrel2_ref[...]) + brel2 + _mm(h1, wroot2_ref[...]),
                0.0)
            pre = (_mm(x, wex_ref[...]) + _mm(h1, weh1_ref[...])
                   + _mm(h2, weh2_ref[...]) + bemb)
            nrm = jnp.sqrt(jnp.sum(pre * pre, axis=1, keepdims=True))
            return pre / jnp.maximum(nrm, 1e-12)

        es = embed(xs_ref[...], h1s_ref[...], a2s_ref[...])
        et = embed(xt_ref[...], h1t_ref[...], a2t_ref[...])
        virt = virt_ref[...]
        vn = virt / jnp.maximum(
            jnp.sqrt(jnp.sum(virt * virt, axis=1, keepdims=True)), 1e-12)

        a2 = jnp.sum(es * es, axis=1, keepdims=True)
        b2 = jnp.sum(et * et, axis=1)
        g = lax.dot_general(es, et, (((1,), (1,)), ((), ())),
                            precision=_HI, preferred_element_type=jnp.float32)
        d2 = a2 + b2[None, :] - 2.0 * g
        c_real = jnp.sqrt(jnp.clip(d2, 1e-12, None))
        gv = lax.dot_general(es, vn, (((1,), (1,)), ((), ())),
                             precision=_HI, preferred_element_type=jnp.float32)
        d2v = a2 + jnp.sum(vn * vn) - 2.0 * gv
        cv = jnp.sqrt(jnp.clip(d2v, 1e-12, None))
        m = jnp.concatenate([c_real, jnp.broadcast_to(cv, (ns, ns - nt))],
                            axis=1)
        m_ref[...] = m

        k = jnp.exp(m * (-1.0 / _REG))

        def upd(u):
            ktu = lax.dot_general(u, k, (((1,), (0,)), ((), ())),
                                  precision=_HI,
                                  preferred_element_type=jnp.float32)
            vv = 1.0 / ktu
            kv = lax.dot_general(vv, k, (((1,), (1,)), ((), ())),
                                 precision=_HI,
                                 preferred_element_type=jnp.float32)
            return 1.0 / kv, vv

        u0 = jnp.full((1, ns), 1.0 / ns, jnp.float32)
        u1, v1 = upd(u0)
        ktu1 = lax.dot_general(u1, k, (((1,), (0,)), ((), ())),
                               precision=_HI, preferred_element_type=jnp.float32)
        tmp2 = v1 * ktu1
        err = jnp.sqrt(jnp.sum((tmp2 - 1.0) ** 2))

        km = k * m

        def geds(u, vv):
            kmv = lax.dot_general(vv, km, (((1,), (1,)), ((), ())),
                                  precision=_HI,
                                  preferred_element_type=jnp.float32)
            return jnp.sum(u * kmv)

        g1 = geds(u1, v1)
        u, vv = u1, v1
        for _ in range(1, _NUM_ITER):
            u, vv = upd(u)
        g8 = geds(u, vv)

        err_ref[...] = jnp.full((1, 1, 128), err, jnp.float32)
        g1_ref[...] = jnp.full((1, 1, 128), g1, jnp.float32)
        g8_ref[...] = jnp.full((1, 1, 128), g8, jnp.float32)

    return body


def _tc_pairs(xs, h1s, a2s, xt, h1t, a2t,
              w_rel2, b_rel2, w_root2, we_x, we_h1, we_h2, b_emb, virt,
              npairs, ns, nt):
    din = xs.shape[1]
    h = h1s.shape[1]
    de = we_x.shape[1]
    body = _pair_kernel(ns, nt, din, h, de, npairs)
    m, err, g1, g8 = pl.pallas_call(
        body,
        grid=(npairs,),
        in_specs=[
            pl.BlockSpec((ns, din), lambda p: (p, 0)),
            pl.BlockSpec((ns, h), lambda p: (p, 0)),
            pl.BlockSpec((2, ns, h), lambda p: (0, p, 0)),
            pl.BlockSpec((nt, din), lambda p: (p, 0)),
            pl.BlockSpec((nt, h), lambda p: (p, 0)),
            pl.BlockSpec((2, nt, h), lambda p: (0, p, 0)),
            pl.BlockSpec((h, h), lambda p: (0, 0)),
            pl.BlockSpec((1, h), lambda p: (0, 0)),
            pl.BlockSpec((h, h), lambda p: (0, 0)),
            pl.BlockSpec((din, de), lambda p: (0, 0)),
            pl.BlockSpec((h, de), lambda p: (0, 0)),
            pl.BlockSpec((h, de), lambda p: (0, 0)),
            pl.BlockSpec((1, de), lambda p: (0, 0)),
            pl.BlockSpec((1, de), lambda p: (0, 0)),
        ],
        out_specs=[
            pl.BlockSpec((ns, ns), lambda p: (p, 0)),
            pl.BlockSpec((1, 1, 128), lambda p: (p, 0, 0)),
            pl.BlockSpec((1, 1, 128), lambda p: (p, 0, 0)),
            pl.BlockSpec((1, 1, 128), lambda p: (p, 0, 0)),
        ],
        out_shape=[
            jax.ShapeDtypeStruct((npairs * ns, ns), jnp.float32),
            jax.ShapeDtypeStruct((npairs, 1, 128), jnp.float32),
            jax.ShapeDtypeStruct((npairs, 1, 128), jnp.float32),
            jax.ShapeDtypeStruct((npairs, 1, 128), jnp.float32),
        ],
    )(xs, h1s, a2s, xt, h1t, a2t,
      w_rel2, b_rel2.reshape(1, h), w_root2,
      we_x, we_h1, we_h2, b_emb.reshape(1, de), virt.reshape(1, de))
    return m, err[:, 0, 0], g1[:, 0, 0], g8[:, 0, 0]


def kernel(x_s, edge_index_s, edge_attr_s, x_t, edge_index_t, edge_attr_t,
           x_s_batch, len_s, len_t, W_rel1, b_rel1, W_root1,
           W_rel2, b_rel2, W_root2, W_emb, b_emb, virtual_embedding):
    npairs = len_s.shape[0]
    n_s, din = x_s.shape
    n_t = x_t.shape[0]
    ns = n_s // npairs
    nt = n_t // npairs
    h = W_rel1.shape[1]
    de = W_emb.shape[1]

    blk = _NSUBCH * _CHUNK
    e_s = edge_index_s.shape[1]
    e_t = edge_index_t.shape[1]
    e_t_pad = ((e_t + blk * _NW - 1) // (blk * _NW)) * (blk * _NW)
    pad_t = e_t_pad - e_t
    src_s = edge_index_s[0].reshape(-1, _NSUBCH, _CHUNK)
    dst_s = edge_index_s[1].reshape(-1, _NSUBCH, _CHUNK)
    src_t = jnp.concatenate(
        [edge_index_t[0], jnp.zeros((pad_t,), edge_index_t.dtype)]
    ).reshape(-1, _NSUBCH, _CHUNK)
    trash = n_t + jnp.arange(pad_t, dtype=edge_index_t.dtype) % (n_s - n_t)
    dst_t = jnp.concatenate(
        [edge_index_t[1], trash]
    ).reshape(-1, _NSUBCH, _CHUNK)

    pad = 128 - din
    x_s_pad = jnp.pad(x_s, ((0, 0), (0, pad)))
    x_t_pad = jnp.pad(x_t, ((0, 0), (0, pad)))
    w_rel1_pad = jnp.pad(W_rel1, ((0, pad), (0, 0)))

    agg1_s, agg1_t = _sc_segsum2(x_s_pad, src_s, dst_s, x_t_pad, src_t, dst_t)
    h1s = _tc_h1(x_s, agg1_s, w_rel1_pad, b_rel1, W_root1, ns)
    h1t = _tc_h1(x_t, agg1_t, w_rel1_pad, b_rel1, W_root1, nt)

    agg2_s, agg2_t = _sc_segsum2(h1s, src_s, dst_s, h1t, src_t, dst_t)

    we_x = W_emb[:din]
    we_h1 = W_emb[din:din + h]
    we_h2 = W_emb[din + h:]

    m, err, g1, g8 = _tc_pairs(
        x_s, h1s, agg2_s, x_t, h1t, agg2_t,
        W_rel2, b_rel2, W_root2, we_x, we_h1, we_h2, b_emb,
        virtual_embedding, npairs, ns, nt)

    edit_costs2 = m.reshape(npairs, ns, ns)
    done = jnp.all(err < _STOP_THR)
    geds = jnp.where(done, g1, g8)
    geds2 = geds / (len_s + len_t).astype(jnp.float32)
    return (edit_costs2, geds2)


# speedup vs baseline: 2.9032x; 2.9014x over previous
import functools

import jax
import jax.numpy as jnp
from jax import lax
from jax.experimental import pallas as pl
from jax.experimental.pallas import tpu as pltpu
from jax.experimental.pallas import tpu_sc as plsc

_REG = 0.1
_NUM_ITER = 8
_STOP_THR = 1e-6

_NC = 2
_NSUB = 16
_NW = _NC * _NSUB
_CHUNK = 128
_NSUBCH = 4

_HI = jax.lax.Precision.HIGHEST


def _sc_segsum2(xa, srca, dsta, xb, srcb, dstb):
    na, d = xa.shape
    nb = xb.shape[0]
    nmax = max(na, nb)
    nbufs = 2
    n_sub = _NSUBCH

    mesh = plsc.VectorSubcoreMesh(core_axis_name="c", subcore_axis_name="s",
                                  num_cores=_NC, num_subcores=_NSUB)

    def body(xa_hbm, srca_hbm, dsta_hbm, xb_hbm, srcb_hbm, dstb_hbm,
             zeros_a_hbm, zeros_b_hbm, outa_hbm, outb_hbm,
             idx_s_v, idx_d_v, rows0_v, rows1_v, rows2_v,
             acc_sh, gsem, ssem):
        rows_bufs = (rows0_v, rows1_v, rows2_v)
        c = lax.axis_index("c")
        s = lax.axis_index("s")

        def phase(x_hbm, src_hbm, dst_hbm, zeros_hbm, out_hbm, n, e):
            rpt = n // _NSUB
            epw = e // _NW
            r0 = s * rpt
            pltpu.sync_copy(zeros_hbm, acc_sh.at[pl.ds(r0, rpt)])
            plsc.subcore_barrier()

            w = c * _NSUB + s
            n_outer = epw // (n_sub * _CHUNK)
            row_base = w * (epw // _CHUNK)

            def outer(j, carry):
                rb = row_base + j * n_sub
                pltpu.sync_copy(src_hbm.at[pl.ds(rb, n_sub)], idx_s_v)
                pltpu.sync_copy(dst_hbm.at[pl.ds(rb, n_sub)], idx_d_v)
                gd = [None] * n_sub
                sd = [None] * n_sub
                for k in range(min(nbufs - 1, n_sub)):
                    gd[k] = pltpu.async_copy(x_hbm.at[idx_s_v.at[k]],
                                             rows_bufs[k % nbufs], gsem)
                for k in range(n_sub):
                    buf = rows_bufs[k % nbufs]
                    gd[k].wait()
                    sd[k] = pltpu.async_copy(buf, acc_sh.at[idx_d_v.at[k]],
                                             ssem, add=True)
                    kn = k + nbufs - 1
                    if kn < n_sub:
                        if kn - nbufs >= 0:
                            sd[kn - nbufs].wait()
                        gd[kn] = pltpu.async_copy(
                            x_hbm.at[idx_s_v.at[kn]],
                            rows_bufs[kn % nbufs], gsem)
                for k in range(max(0, n_sub - nbufs), n_sub):
                    if sd[k] is not None:
                        sd[k].wait()
                return carry

            lax.fori_loop(0, n_outer, outer, 0)
            plsc.subcore_barrier()
            pltpu.sync_copy(acc_sh.at[pl.ds(r0, rpt)],
                            out_hbm.at[pl.ds(c * n + r0, rpt)])
            plsc.subcore_barrier()

        ea = srca_hbm.shape[0] * _CHUNK
        eb = srcb_hbm.shape[0] * _CHUNK
        phase(xa_hbm, srca_hbm, dsta_hbm, zeros_a_hbm, outa_hbm, na, ea)
        phase(xb_hbm, srcb_hbm, dstb_hbm, zeros_b_hbm, outb_hbm, nb, eb)

    kfn = pl.kernel(
        body,
        out_type=(jax.ShapeDtypeStruct((2 * na, d), jnp.float32),
                  jax.ShapeDtypeStruct((2 * nb, d), jnp.float32)),
        mesh=mesh,
        scratch_types=[
            pltpu.VMEM((_NSUBCH, _CHUNK), jnp.int32),
            pltpu.VMEM((_NSUBCH, _CHUNK), jnp.int32),
            pltpu.VMEM((_CHUNK, d), jnp.float32),
            pltpu.VMEM((_CHUNK, d), jnp.float32),
            pltpu.VMEM((_CHUNK, d), jnp.float32),
            pltpu.VMEM_SHARED((nmax, d), jnp.float32),
            pltpu.SemaphoreType.DMA,
            pltpu.SemaphoreType.DMA,
        ],
    )
    zeros_a = jnp.zeros((na // _NSUB, d), jnp.float32)
    zeros_b = jnp.zeros((nb // _NSUB, d), jnp.float32)
    outa, outb = kfn(xa, srca, dsta, xb, srcb, dstb, zeros_a, zeros_b)
    return outa.reshape(2, na, d), outb.reshape(2, nb, d)


def _tc_h1(xg, parts, w_rel, b_rel, w_root, blk):
    N, din = xg.shape
    da = parts.shape[2]
    h = w_rel.shape[1]
    grid = N // blk

    def body(x_ref, p_ref, wrel_ref, brel_ref, wroot_ref, out_ref):
        agg = p_ref[0] + p_ref[1]
        acc = lax.dot_general(agg, wrel_ref[...], (((1,), (0,)), ((), ())),
                              precision=_HI, preferred_element_type=jnp.float32)
        acc = acc + lax.dot_general(x_ref[...], wroot_ref[...],
                                    (((1,), (0,)), ((), ())),
                                    precision=_HI,
                                    preferred_element_type=jnp.float32)
        out_ref[...] = jnp.maximum(acc + brel_ref[...], 0.0)

    return pl.pallas_call(
        body,
        grid=(grid,),
        in_specs=[
            pl.BlockSpec((blk, din), lambda p: (p, 0)),
            pl.BlockSpec((2, blk, da), lambda p: (0, p, 0)),
            pl.BlockSpec((da, h), lambda p: (0, 0)),
            pl.BlockSpec((1, h), lambda p: (0, 0)),
            pl.BlockSpec((din, h), lambda p: (0, 0)),
        ],
        out_specs=pl.BlockSpec((blk, h), lambda p: (p, 0)),
        out_shape=jax.ShapeDtypeStruct((N, h), jnp.float32),
    )(xg, parts, w_rel, b_rel.reshape(1, h), w_root)


def _mm(a, b):
    return lax.dot_general(a, b, (((1,), (0,)), ((), ())),
                           precision=_HI, preferred_element_type=jnp.float32)


def _pair_kernel(ns, nt, din, h, de, npairs):
    def body(xs_ref, h1s_ref, a2s_ref, xt_ref, h1t_ref, a2t_ref,
             wrel2_ref, brel2_ref, wroot2_ref,
             wex_ref, weh1_ref, weh2_ref, bemb_ref, virt_ref,
             m_ref, err_ref, g1_ref, g8_ref):
        brel2 = brel2_ref[...]
        bemb = bemb_ref[...]

        def embed(x, h1, aggp):
            agg = aggp[0] + aggp[1]
            h2 = jnp.maximum(
                _mm(agg, wrel2_ref[...]) + brel2 + _mm(h1, wroot2_ref[...]),
                0.0)
            pre = (_mm(x, wex_ref[...]) + _mm(h1, weh1_ref[...])
                   + _mm(h2, weh2_ref[...]) + bemb)
            nrm = jnp.sqrt(jnp.sum(pre * pre, axis=1, keepdims=True))
            return pre / jnp.maximum(nrm, 1e-12)

        es = embed(xs_ref[...], h1s_ref[...], a2s_ref[...])
        et = embed(xt_ref[...], h1t_ref[...], a2t_ref[...])
        virt = virt_ref[...]
        vn = virt / jnp.maximum(
            jnp.sqrt(jnp.sum(virt * virt, axis=1, keepdims=True)), 1e-12)

        a2 = jnp.sum(es * es, axis=1, keepdims=True)
        b2 = jnp.sum(et * et, axis=1)
        g = lax.dot_general(es, et, (((1,), (1,)), ((), ())),
                            precision=_HI, preferred_element_type=jnp.float32)
        d2 = a2 + b2[None, :] - 2.0 * g
        c_real = jnp.sqrt(jnp.clip(d2, 1e-12, None))
        gv = lax.dot_general(es, vn, (((1,), (1,)), ((), ())),
                             precision=_HI, preferred_element_type=jnp.float32)
        d2v = a2 + jnp.sum(vn * vn) - 2.0 * gv
        cv = jnp.sqrt(jnp.clip(d2v, 1e-12, None))
        m = jnp.concatenate([c_real, jnp.broadcast_to(cv, (ns, ns - nt))],
                            axis=1)
        m_ref[...] = m

        k = jnp.exp(m * (-1.0 / _REG))

        def upd(u):
            ktu = lax.dot_general(u, k, (((1,), (0,)), ((), ())),
                                  precision=_HI,
                                  preferred_element_type=jnp.float32)
            vv = 1.0 / ktu
            kv = lax.dot_general(vv, k, (((1,), (1,)), ((), ())),
                                 precision=_HI,
                                 preferred_element_type=jnp.float32)
            return 1.0 / kv, vv

        u0 = jnp.full((1, ns), 1.0 / ns, jnp.float32)
        u1, v1 = upd(u0)
        ktu1 = lax.dot_general(u1, k, (((1,), (0,)), ((), ())),
                               precision=_HI, preferred_element_type=jnp.float32)
        tmp2 = v1 * ktu1
        err = jnp.sqrt(jnp.sum((tmp2 - 1.0) ** 2))

        km = k * m

        def geds(u, vv):
            kmv = lax.dot_general(vv, km, (((1,), (1,)), ((), ())),
                                  precision=_HI,
                                  preferred_element_type=jnp.float32)
            return jnp.sum(u * kmv)

        g1 = geds(u1, v1)
        u, vv = u1, v1
        for _ in range(1, _NUM_ITER):
            u, vv = upd(u)
        g8 = geds(u, vv)

        err_ref[...] = jnp.full((1, 1, 128), err, jnp.float32)
        g1_ref[...] = jnp.full((1, 1, 128), g1, jnp.float32)
        g8_ref[...] = jnp.full((1, 1, 128), g8, jnp.float32)

    return body


def _tc_pairs(xs, h1s, a2s, xt, h1t, a2t,
              w_rel2, b_rel2, w_root2, we_x, we_h1, we_h2, b_emb, virt,
              npairs, ns, nt):
    din = xs.shape[1]
    h = h1s.shape[1]
    de = we_x.shape[1]
    body = _pair_kernel(ns, nt, din, h, de, npairs)
    m, err, g1, g8 = pl.pallas_call(
        body,
        grid=(npairs,),
        in_specs=[
            pl.BlockSpec((ns, din), lambda p: (p, 0)),
            pl.BlockSpec((ns, h), lambda p: (p, 0)),
            pl.BlockSpec((2, ns, h), lambda p: (0, p, 0)),
            pl.BlockSpec((nt, din), lambda p: (p, 0)),
            pl.BlockSpec((nt, h), lambda p: (p, 0)),
            pl.BlockSpec((2, nt, h), lambda p: (0, p, 0)),
            pl.BlockSpec((h, h), lambda p: (0, 0)),
            pl.BlockSpec((1, h), lambda p: (0, 0)),
            pl.BlockSpec((h, h), lambda p: (0, 0)),
            pl.BlockSpec((din, de), lambda p: (0, 0)),
            pl.BlockSpec((h, de), lambda p: (0, 0)),
            pl.BlockSpec((h, de), lambda p: (0, 0)),
            pl.BlockSpec((1, de), lambda p: (0, 0)),
            pl.BlockSpec((1, de), lambda p: (0, 0)),
        ],
        out_specs=[
            pl.BlockSpec((ns, ns), lambda p: (p, 0)),
            pl.BlockSpec((1, 1, 128), lambda p: (p, 0, 0)),
            pl.BlockSpec((1, 1, 128), lambda p: (p, 0, 0)),
            pl.BlockSpec((1, 1, 128), lambda p: (p, 0, 0)),
        ],
        out_shape=[
            jax.ShapeDtypeStruct((npairs * ns, ns), jnp.float32),
            jax.ShapeDtypeStruct((npairs, 1, 128), jnp.float32),
            jax.ShapeDtypeStruct((npairs, 1, 128), jnp.float32),
            jax.ShapeDtypeStruct((npairs, 1, 128), jnp.float32),
        ],
    )(xs, h1s, a2s, xt, h1t, a2t,
      w_rel2, b_rel2.reshape(1, h), w_root2,
      we_x, we_h1, we_h2, b_emb.reshape(1, de), virt.reshape(1, de))
    return m, err[:, 0, 0], g1[:, 0, 0], g8[:, 0, 0]


def kernel(x_s, edge_index_s, edge_attr_s, x_t, edge_index_t, edge_attr_t,
           x_s_batch, len_s, len_t, W_rel1, b_rel1, W_root1,
           W_rel2, b_rel2, W_root2, W_emb, b_emb, virtual_embedding):
    npairs = len_s.shape[0]
    n_s, din = x_s.shape
    n_t = x_t.shape[0]
    ns = n_s // npairs
    nt = n_t // npairs
    h = W_rel1.shape[1]
    de = W_emb.shape[1]

    src_s = edge_index_s[0].reshape(-1, _CHUNK)
    dst_s = edge_index_s[1].reshape(-1, _CHUNK)
    src_t = edge_index_t[0].reshape(-1, _CHUNK)
    dst_t = edge_index_t[1].reshape(-1, _CHUNK)

    pad = 128 - din
    x_s_pad = jnp.pad(x_s, ((0, 0), (0, pad)))
    x_t_pad = jnp.pad(x_t, ((0, 0), (0, pad)))
    w_rel1_pad = jnp.pad(W_rel1, ((0, pad), (0, 0)))

    agg1_s, agg1_t = _sc_segsum2(x_s_pad, src_s, dst_s, x_t_pad, src_t, dst_t)
    h1s = _tc_h1(x_s, agg1_s, w_rel1_pad, b_rel1, W_root1, ns)
    h1t = _tc_h1(x_t, agg1_t, w_rel1_pad, b_rel1, W_root1, nt)

    agg2_s, agg2_t = _sc_segsum2(h1s, src_s, dst_s, h1t, src_t, dst_t)

    we_x = W_emb[:din]
    we_h1 = W_emb[din:din + h]
    we_h2 = W_emb[din + h:]

    m, err, g1, g8 = _tc_pairs(
        x_s, h1s, agg2_s, x_t, h1t, agg2_t,
        W_rel2, b_rel2, W_root2, we_x, we_h1, we_h2, b_emb,
        virtual_embedding, npairs, ns, nt)

    edit_costs2 = m.reshape(npairs, ns, ns)
    done = jnp.all(err < _STOP_THR)
    geds = jnp.where(done, g1, g8)
    geds2 = geds / (len_s + len_t).astype(jnp.float32)
    return (edit_costs2, geds2)
